# Initial kernel scaffold; baseline (speedup 1.0000x reference)
import jax
import jax.numpy as jnp
from jax.experimental import pallas as pl


def kernel(h, edge_index, edge_index_knn, W1, b1, W2, b2, Wg0, bg0, Wg1, bg1, Wk0, bk0, Wk1, bk1):
    raise NotImplementedError("write your pallas kernel here")


# speedup vs baseline: 9.5094x; 9.5094x over previous
import dataclasses
import functools

import jax
import jax.numpy as jnp
from jax import lax
from jax.experimental import pallas as pl
from jax.experimental.pallas import tpu as pltpu
from jax.experimental.pallas import tpu_sc as plsc

EPS_C = 0.1
KW_C = 0.5
NC = 2
NS = 16
NW = NC * NS
LANES = 16
CHUNK = 128


def _sc_params():
    cp = pltpu.CompilerParams()
    if "needs_layout_passes" in pltpu.CompilerParams.__dataclass_fields__:
        cp = dataclasses.replace(cp, needs_layout_passes=False)
    return cp


def _round_up(v, m):
    return (v + m - 1) // m * m


def _tc_dense_x(hp, W1, b1, npad):
    br = 1024

    def body(h_ref, w1_ref, b1_ref, x_ref):
        x = lax.dot_general(h_ref[...], w1_ref[...], (((1,), (1,)), ((), ())),
                            preferred_element_type=jnp.float32) + b1_ref[...]
        x_ref[...] = jnp.maximum(x, 0.0)

    return pl.pallas_call(
        body,
        grid=(npad // br,),
        in_specs=[
            pl.BlockSpec((br, 128), lambda i: (i, 0)),
            pl.BlockSpec((128, 128), lambda i: (0, 0)),
            pl.BlockSpec((1, 128), lambda i: (0, 0)),
        ],
        out_specs=pl.BlockSpec((br, 128), lambda i: (i, 0)),
        out_shape=jax.ShapeDtypeStruct((npad, 128), jnp.float32),
    )(hp, W1, b1.reshape(1, 128))


def _table_rows(x, g6_ref, b6_ref, dg_ref):
    s = lax.dot_general(g6_ref[...], x, (((1,), (1,)), ((), ())),
                        preferred_element_type=jnp.float32) + b6_ref[...]
    deg = jnp.sum(dg_ref[...], axis=1)
    dis = lax.rsqrt(jnp.maximum(deg, 1.0))
    return s, dis


def _tc_tables(x, degs, G6, bias6, npad):
    bn = 2048

    def body(x_ref, dg_ref, g6_ref, b6_ref, t_ref):
        s, dis = _table_rows(x_ref[...], g6_ref, b6_ref, dg_ref)
        t_ref[0:2] = s[0:2]
        t_ref[2:3] = dis[0:1]
        t_ref[3:5] = s[3:5]
        t_ref[5:6] = dis[1:2] * KW_C ** 0.5

    return pl.pallas_call(
        body,
        grid=(npad // bn,),
        in_specs=[
            pl.BlockSpec((bn, 128), lambda i: (i, 0)),
            pl.BlockSpec((2, NW, bn), lambda i: (0, 0, i)),
            pl.BlockSpec((6, 128), lambda i: (0, 0)),
            pl.BlockSpec((6, 1), lambda i: (0, 0)),
        ],
        out_specs=pl.BlockSpec((6, bn), lambda i: (0, i)),
        out_shape=jax.ShapeDtypeStruct((6, npad), jnp.float32),
    )(x, degs, G6, bias6)


def _tc_combine_tables(p, xr, degs, G6, bias6, npad):
    bn = 2048

    def body(p_ref, x_ref, dg_ref, g6_ref, b6_ref, x1_ref, t_ref):
        x1 = EPS_C * x_ref[...] + p_ref[0] + p_ref[1]
        x1_ref[...] = x1
        s, dis = _table_rows(x1, g6_ref, b6_ref, dg_ref)
        t_ref[0:2] = s[0:2]
        t_ref[2:3] = dis[0:1]
        t_ref[3:5] = s[3:5]
        t_ref[5:6] = dis[1:2] * KW_C ** 0.5

    return pl.pallas_call(
        body,
        grid=(npad // bn,),
        in_specs=[
            pl.BlockSpec((2, bn, 128), lambda i: (0, i, 0)),
            pl.BlockSpec((bn, 128), lambda i: (i, 0)),
            pl.BlockSpec((2, NW, bn), lambda i: (0, 0, i)),
            pl.BlockSpec((6, 128), lambda i: (0, 0)),
            pl.BlockSpec((6, 1), lambda i: (0, 0)),
        ],
        out_specs=[
            pl.BlockSpec((bn, 128), lambda i: (i, 0)),
            pl.BlockSpec((6, bn), lambda i: (0, i)),
        ],
        out_shape=[
            jax.ShapeDtypeStruct((npad, 128), jnp.float32),
            jax.ShapeDtypeStruct((6, npad), jnp.float32),
        ],
    )(p, xr, degs, G6, bias6)


def _tc_final(q, xr, W2, b2, n, c):
    br = 1000

    def body(q_ref, x_ref, w2_ref, b2_ref, o_ref):
        x2 = EPS_C * x_ref[...] + q_ref[0] + q_ref[1]
        logits = lax.dot_general(x2, w2_ref[...], (((1,), (1,)), ((), ())),
                                 preferred_element_type=jnp.float32) + b2_ref[...]
        m = jnp.max(logits, axis=1, keepdims=True)
        lse = jnp.log(jnp.sum(jnp.exp(logits - m), axis=1, keepdims=True)) + m
        o_ref[...] = logits - lse

    return pl.pallas_call(
        body,
        grid=(n // br,),
        in_specs=[
            pl.BlockSpec((2, br, 128), lambda i: (0, i, 0)),
            pl.BlockSpec((br, 128), lambda i: (i, 0)),
            pl.BlockSpec((c, 128), lambda i: (0, 0)),
            pl.BlockSpec((1, c), lambda i: (0, 0)),
        ],
        out_specs=pl.BlockSpec((br, c), lambda i: (i, 0)),
        out_shape=jax.ShapeDtypeStruct((n, c), jnp.float32),
    )(q, xr, W2, b2.reshape(1, c))


def _sc_degree(ro, rk, zeros_n, npad):
    mesh = plsc.VectorSubcoreMesh(core_axis_name="c", subcore_axis_name="s")
    dc_o = ro.shape[0] // NW
    dc_k = rk.shape[0] // NW

    @functools.partial(
        pl.kernel,
        out_type=jax.ShapeDtypeStruct((2, NW, npad), jnp.float32),
        mesh=mesh,
        compiler_params=_sc_params(),
        scratch_types=[
            pltpu.VMEM((npad,), jnp.float32),
            pltpu.VMEM((npad,), jnp.float32),
            pltpu.VMEM((dc_o,), jnp.int32),
            pltpu.VMEM((dc_k,), jnp.int32),
        ],
    )
    def deg_kernel(ro_hbm, rk_hbm, z_hbm, out_hbm, ho_v, hk_v, io_v, ik_v):
        ci = lax.axis_index("c")
        si = lax.axis_index("s")
        wid = ci * NS + si
        pltpu.sync_copy(z_hbm, ho_v)
        pltpu.sync_copy(z_hbm, hk_v)
        pltpu.sync_copy(ro_hbm.at[pl.ds(wid * dc_o, dc_o)], io_v)
        pltpu.sync_copy(rk_hbm.at[pl.ds(wid * dc_k, dc_k)], ik_v)
        ones = jnp.full((LANES,), 1.0, jnp.float32)

        @pl.loop(0, dc_o, step=LANES)
        def _(i):
            plsc.addupdate_scatter(ho_v, [io_v[pl.ds(i, LANES)]], ones)

        @pl.loop(0, dc_k, step=LANES)
        def _(i):
            plsc.addupdate_scatter(hk_v, [ik_v[pl.ds(i, LANES)]], ones)

        pltpu.sync_copy(ho_v, out_hbm.at[0, wid])
        pltpu.sync_copy(hk_v, out_hbm.at[1, wid])

    return deg_kernel(ro, rk, zeros_n)


def _sc_edge_pass(x, t_tab, ro, co, rk, ck, zeros_blk, npad,
                  nt_o, pt_o, pt_k):
    mesh = plsc.VectorSubcoreMesh(core_axis_name="c", subcore_axis_name="s")
    rpt = npad // NS

    @functools.partial(
        pl.kernel,
        out_type=jax.ShapeDtypeStruct((NC, npad, 128), jnp.float32),
        mesh=mesh,
        compiler_params=_sc_params(),
        scratch_types=[
            pltpu.VMEM((3 * npad,), jnp.float32),
            pltpu.VMEM((CHUNK, 128), jnp.float32),
            pltpu.VMEM((CHUNK,), jnp.int32),
            pltpu.VMEM((1, CHUNK), jnp.int32),
            pltpu.VMEM((CHUNK,), jnp.float32),
            pltpu.VMEM_SHARED((npad, 128), jnp.float32),
            pltpu.SemaphoreType.DMA,
        ],
    )
    def edge_kernel(x_hbm, t_hbm, ro_hbm, co_hbm, rk_hbm, ck_hbm, z_hbm,
                    p_hbm, t_v, r_v, row_v, col_v, c_v, acc, sem):
        ci = lax.axis_index("c")
        si = lax.axis_index("s")
        wid = ci * NS + si
        is_org = wid < nt_o
        pltpu.sync_copy(z_hbm, acc.at[pl.ds(si * rpt, rpt)])
        tb = jnp.where(is_org, 0, 3 * npad)
        pltpu.sync_copy(t_hbm.at[pl.ds(tb, 3 * npad)], t_v)
        plsc.subcore_barrier()

        aoff = jnp.full((LANES,), 0, jnp.int32)
        boff = jnp.full((LANES,), npad, jnp.int32)
        doff = jnp.full((LANES,), 2 * npad, jnp.int32)

        def family(r_hbm, c_hbm, per_tile, tid):
            base0 = tid * per_tile

            @pl.loop(0, per_tile // CHUNK)
            def _(i):
                base = base0 + i * CHUNK
                pltpu.sync_copy(r_hbm.at[pl.ds(base, CHUNK)], row_v)
                pltpu.sync_copy(c_hbm.at[pl.ds(base, CHUNK)], col_v.at[0])
                gat = pltpu.async_copy(x_hbm.at[row_v], r_v, sem)
                for g in range(CHUNK // LANES):
                    r16 = row_v[pl.ds(g * LANES, LANES)]
                    c16 = col_v[0, pl.ds(g * LANES, LANES)]
                    a = plsc.load_gather(t_v, [r16 + aoff])
                    b = plsc.load_gather(t_v, [c16 + boff])
                    dr = plsc.load_gather(t_v, [r16 + doff])
                    dc = plsc.load_gather(t_v, [c16 + doff])
                    z2 = (a + b) * 2.0
                    gate = 1.0 - 2.0 / (jnp.exp(z2) + 1.0)
                    c_v[pl.ds(g * LANES, LANES)] = gate * dr * dc
                gat.wait()

                @pl.loop(0, CHUNK)
                def _(e):
                    ev = jnp.full((LANES,), e, jnp.int32)
                    cb = plsc.load_gather(c_v, [ev])
                    for j in range(128 // LANES):
                        sl = pl.ds(j * LANES, LANES)
                        r_v[e, sl] = r_v[e, sl] * cb

                pltpu.sync_copy(r_v, acc.at[col_v.at[0]], add=True)

        @pl.when(is_org)
        def _():
            family(ro_hbm, co_hbm, pt_o, wid)

        @pl.when(jnp.logical_not(is_org))
        def _():
            family(rk_hbm, ck_hbm, pt_k, wid - nt_o)

        plsc.subcore_barrier()
        pltpu.sync_copy(acc.at[pl.ds(si * rpt, rpt)],
                        p_hbm.at[ci, pl.ds(si * rpt, rpt)])

    return edge_kernel(x, t_tab, ro, co, rk, ck, zeros_blk)


def kernel(h, edge_index, edge_index_knn, W1, b1, W2, b2,
           Wg0, bg0, Wg1, bg1, Wk0, bk0, Wk1, bk1):
    n, f = h.shape
    c = W2.shape[0]
    npad = _round_up(n + 1, 2048)
    e_o = edge_index.shape[1]
    e_k = edge_index_knn.shape[1]

    nt_o = max(1, min(NW - 1, int(round(NW * e_o / (e_o + e_k)))))
    nt_k = NW - nt_o
    pt_o = _round_up(-(-e_o // nt_o), CHUNK)
    pt_k = _round_up(-(-e_k // nt_k), CHUNK)

    def pad_idx(a, total):
        return jnp.pad(a, (0, total - a.shape[0]), constant_values=n)

    tot_o = _round_up(nt_o * pt_o, NW * CHUNK)
    tot_k = _round_up(nt_k * pt_k, NW * CHUNK)
    ro, co = pad_idx(edge_index[0], tot_o), pad_idx(edge_index[1], tot_o)
    rk, ck = pad_idx(edge_index_knn[0], tot_k), pad_idx(edge_index_knn[1], tot_k)
    hp = jnp.pad(h, ((0, npad - n), (0, 0)))

    def gate_mat(Wg, bg, Wk, bk):
        zv = jnp.zeros((1, f), jnp.float32)
        G6 = jnp.concatenate(
            [Wg[0:1, :f], Wg[0:1, f:], zv, Wk[0:1, :f], Wk[0:1, f:], zv])
        z1 = jnp.zeros((1,), jnp.float32)
        bias6 = jnp.concatenate([z1, bg, z1, z1, bk, z1]).reshape(6, 1)
        return G6, bias6

    G6_0, bias6_0 = gate_mat(Wg0, bg0, Wk0, bk0)
    G6_1, bias6_1 = gate_mat(Wg1, bg1, Wk1, bk1)

    zeros_n = jnp.zeros((npad,), jnp.float32)
    zeros_blk = jnp.zeros((npad // NS, 128), jnp.float32)

    degs = _sc_degree(ro, rk, zeros_n, npad)
    x = _tc_dense_x(hp, W1, b1, npad)
    t0 = _tc_tables(x, degs, G6_0, bias6_0, npad).reshape(6 * npad)
    p = _sc_edge_pass(x, t0, ro, co, rk, ck, zeros_blk, npad,
                      nt_o, pt_o, pt_k)
    x1, t1 = _tc_combine_tables(p, x, degs, G6_1, bias6_1, npad)
    q = _sc_edge_pass(x1, t1.reshape(6 * npad), ro, co, rk, ck, zeros_blk,
                      npad, nt_o, pt_o, pt_k)
    return _tc_final(q, x, W2, b2, n, c)


# speedup vs baseline: 11.3334x; 1.1918x over previous
import dataclasses
import functools

import jax
import jax.numpy as jnp
from jax import lax
from jax.experimental import pallas as pl
from jax.experimental.pallas import tpu as pltpu
from jax.experimental.pallas import tpu_sc as plsc

EPS_C = 0.1
KW_C = 0.5
NC = 2
NS = 16
NW = NC * NS
LANES = 16
CHUNK = 128


def _sc_params():
    cp = pltpu.CompilerParams()
    if "needs_layout_passes" in pltpu.CompilerParams.__dataclass_fields__:
        cp = dataclasses.replace(cp, needs_layout_passes=False)
    return cp


def _round_up(v, m):
    return (v + m - 1) // m * m


def _tc_dense_x(hp, W1, b1, npad):
    br = 1024

    def body(h_ref, w1_ref, b1_ref, x_ref):
        x = lax.dot_general(h_ref[...], w1_ref[...], (((1,), (1,)), ((), ())),
                            preferred_element_type=jnp.float32) + b1_ref[...]
        x_ref[...] = jnp.maximum(x, 0.0)

    return pl.pallas_call(
        body,
        grid=(npad // br,),
        in_specs=[
            pl.BlockSpec((br, 128), lambda i: (i, 0)),
            pl.BlockSpec((128, 128), lambda i: (0, 0)),
            pl.BlockSpec((1, 128), lambda i: (0, 0)),
        ],
        out_specs=pl.BlockSpec((br, 128), lambda i: (i, 0)),
        out_shape=jax.ShapeDtypeStruct((npad, 128), jnp.float32),
    )(hp, W1, b1.reshape(1, 128))


def _table_rows(x, g6_ref, b6_ref, dg_ref):
    s = lax.dot_general(g6_ref[...], x, (((1,), (1,)), ((), ())),
                        preferred_element_type=jnp.float32) + b6_ref[...]
    deg = jnp.sum(dg_ref[...], axis=1)
    dis = lax.rsqrt(jnp.maximum(deg, 1.0))
    return s, dis


def _tc_tables(x, degs, G6, bias6, npad):
    bn = 2048

    def body(x_ref, dg_ref, g6_ref, b6_ref, t_ref):
        s, dis = _table_rows(x_ref[...], g6_ref, b6_ref, dg_ref)
        t_ref[0:2] = s[0:2]
        t_ref[2:3] = dis[0:1]
        t_ref[3:5] = s[3:5]
        t_ref[5:6] = dis[1:2] * KW_C ** 0.5

    return pl.pallas_call(
        body,
        grid=(npad // bn,),
        in_specs=[
            pl.BlockSpec((bn, 128), lambda i: (i, 0)),
            pl.BlockSpec((2, NW, bn), lambda i: (0, 0, i)),
            pl.BlockSpec((6, 128), lambda i: (0, 0)),
            pl.BlockSpec((6, 1), lambda i: (0, 0)),
        ],
        out_specs=pl.BlockSpec((6, bn), lambda i: (0, i)),
        out_shape=jax.ShapeDtypeStruct((6, npad), jnp.float32),
    )(x, degs, G6, bias6)


def _tc_combine_tables(p, xr, degs, G6, bias6, npad):
    bn = 2048

    def body(p_ref, x_ref, dg_ref, g6_ref, b6_ref, x1_ref, t_ref):
        x1 = EPS_C * x_ref[...] + p_ref[0] + p_ref[1]
        x1_ref[...] = x1
        s, dis = _table_rows(x1, g6_ref, b6_ref, dg_ref)
        t_ref[0:2] = s[0:2]
        t_ref[2:3] = dis[0:1]
        t_ref[3:5] = s[3:5]
        t_ref[5:6] = dis[1:2] * KW_C ** 0.5

    return pl.pallas_call(
        body,
        grid=(npad // bn,),
        in_specs=[
            pl.BlockSpec((2, bn, 128), lambda i: (0, i, 0)),
            pl.BlockSpec((bn, 128), lambda i: (i, 0)),
            pl.BlockSpec((2, NW, bn), lambda i: (0, 0, i)),
            pl.BlockSpec((6, 128), lambda i: (0, 0)),
            pl.BlockSpec((6, 1), lambda i: (0, 0)),
        ],
        out_specs=[
            pl.BlockSpec((bn, 128), lambda i: (i, 0)),
            pl.BlockSpec((6, bn), lambda i: (0, i)),
        ],
        out_shape=[
            jax.ShapeDtypeStruct((npad, 128), jnp.float32),
            jax.ShapeDtypeStruct((6, npad), jnp.float32),
        ],
    )(p, xr, degs, G6, bias6)


def _tc_final(q, xr, W2, b2, n, c):
    br = 1000

    def body(q_ref, x_ref, w2_ref, b2_ref, o_ref):
        x2 = EPS_C * x_ref[...] + q_ref[0] + q_ref[1]
        logits = lax.dot_general(x2, w2_ref[...], (((1,), (1,)), ((), ())),
                                 preferred_element_type=jnp.float32) + b2_ref[...]
        m = jnp.max(logits, axis=1, keepdims=True)
        lse = jnp.log(jnp.sum(jnp.exp(logits - m), axis=1, keepdims=True)) + m
        o_ref[...] = logits - lse

    return pl.pallas_call(
        body,
        grid=(n // br,),
        in_specs=[
            pl.BlockSpec((2, br, 128), lambda i: (0, i, 0)),
            pl.BlockSpec((br, 128), lambda i: (i, 0)),
            pl.BlockSpec((c, 128), lambda i: (0, 0)),
            pl.BlockSpec((1, c), lambda i: (0, 0)),
        ],
        out_specs=pl.BlockSpec((br, c), lambda i: (i, 0)),
        out_shape=jax.ShapeDtypeStruct((n, c), jnp.float32),
    )(q, xr, W2, b2.reshape(1, c))


def _sc_degree(ro, rk, zeros_n, npad):
    mesh = plsc.VectorSubcoreMesh(core_axis_name="c", subcore_axis_name="s")
    dc_o = ro.shape[0] // NW
    dc_k = rk.shape[0] // NW

    @functools.partial(
        pl.kernel,
        out_type=jax.ShapeDtypeStruct((2, NW, npad), jnp.float32),
        mesh=mesh,
        compiler_params=_sc_params(),
        scratch_types=[
            pltpu.VMEM((npad,), jnp.float32),
            pltpu.VMEM((npad,), jnp.float32),
            pltpu.VMEM((dc_o,), jnp.int32),
            pltpu.VMEM((dc_k,), jnp.int32),
        ],
    )
    def deg_kernel(ro_hbm, rk_hbm, z_hbm, out_hbm, ho_v, hk_v, io_v, ik_v):
        ci = lax.axis_index("c")
        si = lax.axis_index("s")
        wid = ci * NS + si
        pltpu.sync_copy(z_hbm, ho_v)
        pltpu.sync_copy(z_hbm, hk_v)
        pltpu.sync_copy(ro_hbm.at[pl.ds(wid * dc_o, dc_o)], io_v)
        pltpu.sync_copy(rk_hbm.at[pl.ds(wid * dc_k, dc_k)], ik_v)
        ones = jnp.full((LANES,), 1.0, jnp.float32)

        @pl.loop(0, dc_o, step=LANES)
        def _(i):
            plsc.addupdate_scatter(ho_v, [io_v[pl.ds(i, LANES)]], ones)

        @pl.loop(0, dc_k, step=LANES)
        def _(i):
            plsc.addupdate_scatter(hk_v, [ik_v[pl.ds(i, LANES)]], ones)

        pltpu.sync_copy(ho_v, out_hbm.at[0, wid])
        pltpu.sync_copy(hk_v, out_hbm.at[1, wid])

    return deg_kernel(ro, rk, zeros_n)


def _sc_edge_pass(x, t_tab, rc_o, rc_k, zeros_blk, npad,
                  nt_o, pt_o, pt_k):
    mesh = plsc.VectorSubcoreMesh(core_axis_name="c", subcore_axis_name="s")
    rpt = npad // NS

    @functools.partial(
        pl.kernel,
        out_type=jax.ShapeDtypeStruct((NC, npad, 128), jnp.float32),
        mesh=mesh,
        compiler_params=_sc_params(),
        scratch_types=[
            pltpu.VMEM((3 * npad,), jnp.float32),
            pltpu.VMEM((CHUNK, 128), jnp.float32),
            pltpu.VMEM((2, CHUNK), jnp.int32),
            pltpu.VMEM((CHUNK,), jnp.float32),
            pltpu.VMEM_SHARED((npad, 128), jnp.float32),
            pltpu.SemaphoreType.DMA,
        ],
    )
    def edge_kernel(x_hbm, t_hbm, rco_hbm, rck_hbm, z_hbm,
                    p_hbm, t_v, r_v, rc_v, c_v, acc, sem):
        ci = lax.axis_index("c")
        si = lax.axis_index("s")
        wid = ci * NS + si
        is_org = wid < nt_o
        pltpu.sync_copy(z_hbm, acc.at[pl.ds(si * rpt, rpt)])
        tb = jnp.where(is_org, 0, 3 * npad)
        pltpu.sync_copy(t_hbm.at[pl.ds(tb, 3 * npad)], t_v)
        plsc.subcore_barrier()

        aoff = jnp.full((LANES,), 0, jnp.int32)
        boff = jnp.full((LANES,), npad, jnp.int32)
        doff = jnp.full((LANES,), 2 * npad, jnp.int32)

        def family(rc_hbm, per_tile, tid):
            nch = per_tile // CHUNK

            @pl.loop(0, nch)
            def _(i):
                pltpu.sync_copy(rc_hbm.at[tid * nch + i], rc_v)
                gat = pltpu.async_copy(x_hbm.at[rc_v.at[0]], r_v, sem)
                for g in range(CHUNK // LANES):
                    r16 = rc_v[0, pl.ds(g * LANES, LANES)]
                    c16 = rc_v[1, pl.ds(g * LANES, LANES)]
                    a = plsc.load_gather(t_v, [r16 + aoff])
                    b = plsc.load_gather(t_v, [c16 + boff])
                    dr = plsc.load_gather(t_v, [r16 + doff])
                    dc = plsc.load_gather(t_v, [c16 + doff])
                    z2 = (a + b) * 2.0
                    gate = 1.0 - 2.0 / (jnp.exp(z2) + 1.0)
                    c_v[pl.ds(g * LANES, LANES)] = gate * dr * dc
                gat.wait()

                @plsc.parallel_loop(0, CHUNK, unroll=4)
                def _(e):
                    ev = jnp.full((LANES,), e, jnp.int32)
                    cb = plsc.load_gather(c_v, [ev])
                    for j in range(128 // LANES):
                        sl = pl.ds(j * LANES, LANES)
                        r_v[e, sl] = r_v[e, sl] * cb

                pltpu.sync_copy(r_v, acc.at[rc_v.at[1]], add=True)

        @pl.when(is_org)
        def _():
            family(rco_hbm, pt_o, wid)

        @pl.when(jnp.logical_not(is_org))
        def _():
            family(rck_hbm, pt_k, wid - nt_o)

        plsc.subcore_barrier()
        pltpu.sync_copy(acc.at[pl.ds(si * rpt, rpt)],
                        p_hbm.at[ci, pl.ds(si * rpt, rpt)])

    return edge_kernel(x, t_tab, rc_o, rc_k, zeros_blk)


def kernel(h, edge_index, edge_index_knn, W1, b1, W2, b2,
           Wg0, bg0, Wg1, bg1, Wk0, bk0, Wk1, bk1):
    n, f = h.shape
    c = W2.shape[0]
    npad = _round_up(n + 1, 2048)
    e_o = edge_index.shape[1]
    e_k = edge_index_knn.shape[1]

    nt_o = max(1, min(NW - 1, int(round(NW * e_o / (e_o + e_k)))))
    nt_k = NW - nt_o
    pt_o = _round_up(-(-e_o // nt_o), CHUNK)
    pt_k = _round_up(-(-e_k // nt_k), CHUNK)

    def pad_idx(a, total):
        return jnp.pad(a, (0, total - a.shape[0]), constant_values=n)

    tot_o = _round_up(nt_o * pt_o, NW * CHUNK)
    tot_k = _round_up(nt_k * pt_k, NW * CHUNK)
    ro, co = pad_idx(edge_index[0], tot_o), pad_idx(edge_index[1], tot_o)
    rk, ck = pad_idx(edge_index_knn[0], tot_k), pad_idx(edge_index_knn[1], tot_k)
    hp = jnp.pad(h, ((0, npad - n), (0, 0)))

    def interleave(r, c, used):
        return jnp.stack([r[:used].reshape(-1, CHUNK),
                          c[:used].reshape(-1, CHUNK)], axis=1)

    rc_o = interleave(ro, co, nt_o * pt_o)
    rc_k = interleave(rk, ck, nt_k * pt_k)

    def gate_mat(Wg, bg, Wk, bk):
        zv = jnp.zeros((1, f), jnp.float32)
        G6 = jnp.concatenate(
            [Wg[0:1, :f], Wg[0:1, f:], zv, Wk[0:1, :f], Wk[0:1, f:], zv])
        z1 = jnp.zeros((1,), jnp.float32)
        bias6 = jnp.concatenate([z1, bg, z1, z1, bk, z1]).reshape(6, 1)
        return G6, bias6

    G6_0, bias6_0 = gate_mat(Wg0, bg0, Wk0, bk0)
    G6_1, bias6_1 = gate_mat(Wg1, bg1, Wk1, bk1)

    zeros_n = jnp.zeros((npad,), jnp.float32)
    zeros_blk = jnp.zeros((npad // NS, 128), jnp.float32)

    degs = _sc_degree(ro, rk, zeros_n, npad)
    x = _tc_dense_x(hp, W1, b1, npad)
    t0 = _tc_tables(x, degs, G6_0, bias6_0, npad).reshape(6 * npad)
    p = _sc_edge_pass(x, t0, rc_o, rc_k, zeros_blk, npad,
                      nt_o, pt_o, pt_k)
    x1, t1 = _tc_combine_tables(p, x, degs, G6_1, bias6_1, npad)
    q = _sc_edge_pass(x1, t1.reshape(6 * npad), rc_o, rc_k, zeros_blk,
                      npad, nt_o, pt_o, pt_k)
    return _tc_final(q, x, W2, b2, n, c)
